# Initial kernel scaffold; baseline (speedup 1.0000x reference)
import jax
import jax.numpy as jnp
from jax.experimental import pallas as pl


def kernel(src_pts, tgt_pts):
    raise NotImplementedError("write your pallas kernel here")


# speedup vs baseline: 26.9004x; 26.9004x over previous
import dataclasses
import functools

import jax
import jax.numpy as jnp
from jax import lax
from jax.experimental import pallas as pl
from jax.experimental.pallas import tpu as pltpu
from jax.experimental.pallas import tpu_sc as plsc

B = 2
Q = 2048
N = 16384
K_NN = 32
L = 16
NCHUNK = N // L
NGROUP = NCHUNK // 16
NSUPER = NGROUP // 16
QPT = Q // 16

_BIG = 1 << 20
_INF = float("inf")


def _bf16_round(x):
  u = lax.bitcast_convert_type(x, jnp.int32)
  rounded = u + 0x7FFF + (lax.shift_right_logical(u, 16) & 1)
  masked = rounded & jnp.int32(-65536)
  return lax.bitcast_convert_type(masked, jnp.float32)


def _sqrt16(x):
  i = lax.bitcast_convert_type(x, jnp.int32)
  i = 0x5F3759DF - lax.shift_right_logical(i, 1)
  y = lax.bitcast_convert_type(i, jnp.float32)
  for _ in range(4):
    y = y * (1.5 - 0.5 * x * y * y)
  return x * y


def _first_match(rows, m):
  found = jnp.full((L,), _BIG, jnp.int32)
  for t, row in enumerate(rows):
    found = jnp.minimum(found, jnp.where(row == m, jnp.int32(t), _BIG))
  return jnp.min(found)


def _sc_body(src_hbm, tgt_hbm, outd_hbm, outi_hbm,
             t_ref, r2_ref, q_ref, d_ref, gmin_ref, smin_ref,
             od_ref, oi_ref, sem):
  c = lax.axis_index("core")
  s = lax.axis_index("subcore")

  pltpu.async_copy(tgt_hbm.at[c], t_ref, sem).wait()
  pltpu.async_copy(src_hbm.at[c], q_ref, sem).wait()

  @pl.loop(0, NCHUNK)
  def _(j):
    tx = t_ref[0, pl.ds(j * L, L)]
    ty = t_ref[1, pl.ds(j * L, L)]
    tz = t_ref[2, pl.ds(j * L, L)]
    r2_ref[pl.ds(j * L, L)] = tx * tx + ty * ty + tz * tz
    t_ref[0, pl.ds(j * L, L)] = _bf16_round(tx)
    t_ref[1, pl.ds(j * L, L)] = _bf16_round(ty)
    t_ref[2, pl.ds(j * L, L)] = _bf16_round(tz)

  lanes = lax.iota(jnp.int32, L)
  qbase = s * QPT

  def _lane_scalar(vec, off):
    return jnp.min(jnp.where(lanes == off, vec, _INF))

  @pl.loop(0, QPT, step=2)
  def _(qi):
    qs = []
    for p in range(2):
      qq = qbase + qi + p
      b16 = qq & (-16)
      off = qq - b16
      qx = _lane_scalar(q_ref[0, pl.ds(b16, L)], off)
      qy = _lane_scalar(q_ref[1, pl.ds(b16, L)], off)
      qz = _lane_scalar(q_ref[2, pl.ds(b16, L)], off)
      q2 = qx * qx + qy * qy + qz * qz
      qs.append((_bf16_round(qx), _bf16_round(qy), _bf16_round(qz), q2))

    @pl.loop(0, NGROUP)
    def _(g):
      gacc = [jnp.full((L,), _INF, jnp.float32) for _ in range(2)]
      for t in range(16):
        j = g * 16 + t
        tx = t_ref[0, pl.ds(j * L, L)]
        ty = t_ref[1, pl.ds(j * L, L)]
        tz = t_ref[2, pl.ds(j * L, L)]
        r2 = r2_ref[pl.ds(j * L, L)]
        for p in range(2):
          qx, qy, qz, q2 = qs[p]
          dot = tx * qx + ty * qy + tz * qz
          key = (q2 + r2) - 2.0 * dot
          d_ref[p, j] = key
          gacc[p] = jnp.minimum(gacc[p], key)
      for p in range(2):
        gmin_ref[p, g] = gacc[p]

    for p in range(2):
      for ss in range(NSUPER):
        acc = jnp.full((L,), _INF, jnp.float32)
        for t in range(16):
          acc = jnp.minimum(acc, gmin_ref[p, ss * 16 + t])
        smin_ref[p, ss] = acc

    def round_body(k, carry):
      new_carry = []
      for p in range(2):
        d0, d1, i0, i1 = carry[p]
        tt = jnp.minimum(jnp.minimum(smin_ref[p, 0], smin_ref[p, 1]),
                         jnp.minimum(smin_ref[p, 2], smin_ref[p, 3]))
        m = jnp.min(tt)
        s_star = _first_match([smin_ref[p, ss] for ss in range(NSUPER)], m)
        g_rel = _first_match([gmin_ref[p, s_star * 16 + t] for t in range(16)], m)
        g_star = s_star * 16 + g_rel
        j_rel = _first_match([d_ref[p, g_star * 16 + t] for t in range(16)], m)
        c_star = g_star * 16 + j_rel
        row = d_ref[p, c_star]
        l_star = jnp.min(plsc.all_reduce_ffs(row == m))
        idx = c_star * L + l_star
        d_ref[p, c_star] = jnp.where(lanes == l_star, _INF, row)
        acc = jnp.full((L,), _INF, jnp.float32)
        for t in range(16):
          acc = jnp.minimum(acc, d_ref[p, g_star * 16 + t])
        gmin_ref[p, g_star] = acc
        acc2 = jnp.full((L,), _INF, jnp.float32)
        for t in range(16):
          acc2 = jnp.minimum(acc2, gmin_ref[p, s_star * 16 + t])
        smin_ref[p, s_star] = acc2
        d0 = jnp.where(lanes == k, m, d0)
        d1 = jnp.where(lanes == k - 16, m, d1)
        i0 = jnp.where(lanes == k, idx, i0)
        i1 = jnp.where(lanes == k - 16, idx, i1)
        new_carry.append((d0, d1, i0, i1))
      return tuple(new_carry)

    init = tuple(
        (jnp.zeros((L,), jnp.float32), jnp.zeros((L,), jnp.float32),
         jnp.zeros((L,), jnp.int32), jnp.zeros((L,), jnp.int32))
        for _ in range(2))
    res = lax.fori_loop(0, K_NN, round_body, init)

    for p in range(2):
      d0, d1, i0, i1 = res[p]
      v0 = _sqrt16(jnp.maximum(d0, 1e-12))
      v1 = _sqrt16(jnp.maximum(d1, 1e-12))
      tot = jnp.sum(v0 + v1)
      od_ref[qi + p, pl.ds(0, L)] = v0 / tot
      od_ref[qi + p, pl.ds(L, L)] = v1 / tot
      oi_ref[qi + p, pl.ds(0, L)] = i0
      oi_ref[qi + p, pl.ds(L, L)] = i1

  pltpu.async_copy(od_ref, outd_hbm.at[c, s], sem).wait()
  pltpu.async_copy(oi_ref, outi_hbm.at[c, s], sem).wait()


@jax.jit
def kernel(src_pts, tgt_pts):
  src_xyz = src_pts[:, :3, :]
  tgt_xyz = tgt_pts[:, :3, :]

  mesh = plsc.VectorSubcoreMesh(core_axis_name="core", subcore_axis_name="subcore")
  cp = pltpu.CompilerParams(use_tc_tiling_on_sc=False)
  if "needs_layout_passes" in pltpu.CompilerParams.__dataclass_fields__:
    cp = dataclasses.replace(cp, needs_layout_passes=False)

  fn = pl.kernel(
      _sc_body,
      out_type=(
          jax.ShapeDtypeStruct((B, 16, QPT, K_NN), jnp.float32),
          jax.ShapeDtypeStruct((B, 16, QPT, K_NN), jnp.int32),
      ),
      mesh=mesh,
      scratch_types=[
          pltpu.VMEM((3, N), jnp.float32),
          pltpu.VMEM((N,), jnp.float32),
          pltpu.VMEM((3, Q), jnp.float32),
          pltpu.VMEM((2, NCHUNK, L), jnp.float32),
          pltpu.VMEM((2, NGROUP, L), jnp.float32),
          pltpu.VMEM((2, NSUPER, L), jnp.float32),
          pltpu.VMEM((QPT, K_NN), jnp.float32),
          pltpu.VMEM((QPT, K_NN), jnp.int32),
          pltpu.SemaphoreType.DMA,
      ],
      compiler_params=cp,
  )
  outd, outi = fn(src_xyz, tgt_xyz)
  return outd.reshape(B, Q, K_NN), outi.reshape(B, Q, K_NN)
